# baseline (speedup 1.0000x reference)
import functools

import jax
import jax.numpy as jnp
from jax import lax
from jax.experimental import pallas as pl
from jax.experimental.pallas import tpu as pltpu
from jax.experimental.pallas import tpu_sc as plsc

_SEQ_LEN = 8192
_N_DIMS = 128
_NUM_CORES = 2
_NUM_SUBCORES = 16
_NUM_WORKERS = _NUM_CORES * _NUM_SUBCORES
_ROWS_PER_W = _SEQ_LEN // _NUM_WORKERS

_mesh = plsc.VectorSubcoreMesh(core_axis_name="c", subcore_axis_name="s")


@functools.partial(
    pl.kernel,
    mesh=_mesh,
    out_type=jax.ShapeDtypeStruct((_SEQ_LEN, _N_DIMS), jnp.float32),
)
def _position_lookup(table_hbm, out_hbm):
    wid = lax.axis_index("s") * _NUM_CORES + lax.axis_index("c")
    base = wid * _ROWS_PER_W
    pltpu.sync_copy(
        table_hbm.at[pl.ds(base, _ROWS_PER_W)],
        out_hbm.at[pl.ds(base, _ROWS_PER_W)],
    )


def kernel(position_embed):
    return _position_lookup(position_embed)[None]


# speedup vs baseline: 6.4798x; 6.4798x over previous
import functools

import jax
import jax.numpy as jnp
from jax import lax
from jax.experimental import pallas as pl
from jax.experimental.pallas import tpu as pltpu
from jax.experimental.pallas import tpu_sc as plsc

_SEQ_LEN = 8192
_N_DIMS = 128
_NUM_CORES = 2
_NUM_SUBCORES = 16
_NUM_WORKERS = _NUM_CORES * _NUM_SUBCORES
_ROWS_PER_W = _SEQ_LEN // _NUM_WORKERS

_mesh = plsc.VectorSubcoreMesh(core_axis_name="c", subcore_axis_name="s")


_NB = 4
_ROWS_SUB = _ROWS_PER_W // _NB


@functools.partial(
    pl.kernel,
    mesh=_mesh,
    out_type=jax.ShapeDtypeStruct((_SEQ_LEN, _N_DIMS), jnp.float32),
    scratch_types=[
        pltpu.VMEM((_ROWS_PER_W, _N_DIMS), jnp.float32),
        pltpu.SemaphoreType.DMA,
        pltpu.SemaphoreType.DMA,
        pltpu.SemaphoreType.DMA,
        pltpu.SemaphoreType.DMA,
        pltpu.SemaphoreType.DMA,
    ],
)
def _position_lookup(table_hbm, out_hbm, buf_v, s0, s1, s2, s3, so):
    wid = lax.axis_index("s") * _NUM_CORES + lax.axis_index("c")
    base = wid * _ROWS_PER_W
    in_sems = (s0, s1, s2, s3)
    loads = []
    for i in range(_NB):
        lo = i * _ROWS_SUB
        loads.append(pltpu.async_copy(
            table_hbm.at[pl.ds(base + lo, _ROWS_SUB)],
            buf_v.at[pl.ds(lo, _ROWS_SUB)],
            in_sems[i]))
    stores = []
    for i in range(_NB):
        lo = i * _ROWS_SUB
        loads[i].wait()
        stores.append(pltpu.async_copy(
            buf_v.at[pl.ds(lo, _ROWS_SUB)],
            out_hbm.at[pl.ds(base + lo, _ROWS_SUB)],
            so))
    for c in stores:
        c.wait()


def kernel(position_embed):
    return _position_lookup(position_embed)[None]


# speedup vs baseline: 6.5468x; 1.0103x over previous
import functools

import jax
import jax.numpy as jnp
from jax import lax
from jax.experimental import pallas as pl
from jax.experimental.pallas import tpu as pltpu
from jax.experimental.pallas import tpu_sc as plsc

_SEQ_LEN = 8192
_N_DIMS = 128
_NUM_CORES = 2
_NUM_SUBCORES = 16
_NUM_WORKERS = _NUM_CORES * _NUM_SUBCORES
_ROWS_PER_W = _SEQ_LEN // _NUM_WORKERS

_mesh = plsc.VectorSubcoreMesh(core_axis_name="c", subcore_axis_name="s")


@functools.partial(
    pl.kernel,
    mesh=_mesh,
    out_type=jax.ShapeDtypeStruct((1, _SEQ_LEN, _N_DIMS), jnp.float32),
    scratch_types=[
        pltpu.VMEM((_ROWS_PER_W, _N_DIMS), jnp.float32),
    ],
)
def _position_lookup(table_hbm, out_hbm, buf_v):
    wid = lax.axis_index("s") * _NUM_CORES + lax.axis_index("c")
    base = wid * _ROWS_PER_W
    pltpu.sync_copy(table_hbm.at[pl.ds(base, _ROWS_PER_W)], buf_v)
    pltpu.sync_copy(buf_v, out_hbm.at[0, pl.ds(base, _ROWS_PER_W)])


def kernel(position_embed):
    return _position_lookup(position_embed)


# speedup vs baseline: 6.5622x; 1.0023x over previous
import functools

import jax
import jax.numpy as jnp
from jax import lax
from jax.experimental import pallas as pl
from jax.experimental.pallas import tpu as pltpu
from jax.experimental.pallas import tpu_sc as plsc

_SEQ_LEN = 8192
_N_DIMS = 128
_NUM_CORES = 2
_NUM_SUBCORES = 16
_NUM_WORKERS = _NUM_CORES * _NUM_SUBCORES
_ROWS_PER_W = _SEQ_LEN // _NUM_WORKERS

_mesh = plsc.VectorSubcoreMesh(core_axis_name="c", subcore_axis_name="s")


@functools.partial(
    pl.kernel,
    mesh=_mesh,
    out_type=jax.ShapeDtypeStruct((1, _SEQ_LEN, _N_DIMS), jnp.float32),
    scratch_types=[
        pltpu.VMEM((_ROWS_PER_W, _N_DIMS), jnp.float32),
    ],
)
def _position_lookup(table_hbm, out_hbm, buf_v):
    wid = lax.axis_index("c") * _NUM_SUBCORES + lax.axis_index("s")
    base = wid * _ROWS_PER_W
    pltpu.sync_copy(table_hbm.at[pl.ds(base, _ROWS_PER_W)], buf_v)
    pltpu.sync_copy(buf_v, out_hbm.at[0, pl.ds(base, _ROWS_PER_W)])


def kernel(position_embed):
    return _position_lookup(position_embed)
